# baseline (speedup 1.0000x reference)
import jax
import jax.numpy as jnp
from jax.experimental import pallas as pl
from jax.experimental.pallas import tpu as pltpu

_LANES = 128


def _make_body(gd, td, m2, have_bias):
    km2 = td * m2
    s = km2 // _LANES

    def body(x_ref, w_ref, b_ref, o_ref):
        for g in range(gd):
            h = jnp.dot(x_ref[:, g, :], w_ref[g],
                        preferred_element_type=jnp.float32)
            if have_bias:
                h = h + b_ref[g].astype(jnp.float32)
            h = h.astype(o_ref.dtype)
            for j in range(s):
                o_ref[:, g * s + j, :] = h[:, j * _LANES:(j + 1) * _LANES]

    def body_nobias(x_ref, w_ref, o_ref):
        body(x_ref, w_ref, None, o_ref)

    return body if have_bias else body_nobias


def kernel(x, weight, bias):
    n, d, m1 = x.shape
    d_w, m1_w, m2 = weight.shape
    assert d == d_w and m1 == m1_w
    assert _LANES % m1 == 0
    td = _LANES // m1
    assert d % td == 0
    gd = d // td
    km2 = td * m2
    assert km2 % _LANES == 0
    s = km2 // _LANES

    tn = 512
    if n <= tn:
        tn = max(8, (n // 8) * 8)
    gn = pl.cdiv(n, tn)
    n_pad = gn * tn
    if n_pad != n:
        x = jnp.pad(x, ((0, n_pad - n), (0, 0), (0, 0)))

    x3 = x.reshape(n_pad, gd, _LANES)

    eye = jnp.eye(td, dtype=weight.dtype)
    w_bd = jnp.einsum('gtio,ts->gtiso', weight.reshape(gd, td, m1, m2), eye)
    w_bd = w_bd.reshape(gd, td * m1, km2)

    x_spec = pl.BlockSpec((tn, gd, _LANES), lambda jn: (jn, 0, 0))
    w_spec = pl.BlockSpec((gd, td * m1, km2), lambda jn: (0, 0, 0))
    o_spec = pl.BlockSpec((tn, gd * s, _LANES), lambda jn: (jn, 0, 0))
    out_shape = jax.ShapeDtypeStruct((n_pad, gd * s, _LANES), x.dtype)

    itemsize = jnp.dtype(x.dtype).itemsize
    cost = pl.CostEstimate(
        flops=int(2 * n_pad * d * td * m1 * m2),
        transcendentals=0,
        bytes_accessed=int((x3.size + w_bd.size + n_pad * d * m2
                            + (d * m2 if bias is not None else 0)) * itemsize),
    )
    cparams = pltpu.CompilerParams(
        dimension_semantics=("parallel",),
        vmem_limit_bytes=100 * 1024 * 1024,
    )

    body = _make_body(gd, td, m2, bias is not None)
    if bias is not None:
        b_bd = bias.reshape(gd, 1, km2)
        b_spec = pl.BlockSpec((gd, 1, km2), lambda jn: (0, 0, 0))
        out3 = pl.pallas_call(
            body,
            out_shape=out_shape,
            grid=(gn,),
            in_specs=[x_spec, w_spec, b_spec],
            out_specs=o_spec,
            compiler_params=cparams,
            cost_estimate=cost,
        )(x3, w_bd, b_bd)
    else:
        out3 = pl.pallas_call(
            body,
            out_shape=out_shape,
            grid=(gn,),
            in_specs=[x_spec, w_spec],
            out_specs=o_spec,
            compiler_params=cparams,
            cost_estimate=cost,
        )(x3, w_bd)

    return out3.reshape(n_pad, d, m2)[:n]


# speedup vs baseline: 3.1844x; 3.1844x over previous
import jax
import jax.numpy as jnp
from jax.experimental import pallas as pl
from jax.experimental.pallas import tpu as pltpu


def _make_body(m1, have_bias):
    def body(x_ref, w_ref, b_ref, o_ref):
        h = x_ref[:, 0:1, :] * w_ref[0][None]
        for i in range(1, m1):
            h = h + x_ref[:, i:i + 1, :] * w_ref[i][None]
        if have_bias:
            h = h + b_ref[...][None]
        o_ref[...] = h.astype(o_ref.dtype)

    def body_nobias(x_ref, w_ref, o_ref):
        body(x_ref, w_ref, None, o_ref)

    return body if have_bias else body_nobias


def kernel(x, weight, bias):
    n, d, m1 = x.shape
    d_w, m1_w, m2 = weight.shape
    assert d == d_w and m1 == m1_w

    tn = 512
    if n <= tn:
        tn = max(8, (n // 8) * 8)
    gn = pl.cdiv(n, tn)
    n_pad = gn * tn
    if n_pad != n:
        x = jnp.pad(x, ((0, n_pad - n), (0, 0), (0, 0)))

    x_t = jnp.transpose(x, (0, 2, 1))
    w_t = jnp.transpose(weight, (1, 2, 0))

    x_spec = pl.BlockSpec((tn, m1, d), lambda jn: (jn, 0, 0))
    w_spec = pl.BlockSpec((m1, m2, d), lambda jn: (0, 0, 0))
    o_spec = pl.BlockSpec((tn, m2, d), lambda jn: (jn, 0, 0))
    out_shape = jax.ShapeDtypeStruct((n_pad, m2, d), x.dtype)

    itemsize = jnp.dtype(x.dtype).itemsize
    cost = pl.CostEstimate(
        flops=int(2 * n_pad * d * m1 * m2),
        transcendentals=0,
        bytes_accessed=int((x_t.size + w_t.size + n_pad * d * m2
                            + (d * m2 if bias is not None else 0)) * itemsize),
    )
    cparams = pltpu.CompilerParams(
        dimension_semantics=("parallel",),
        vmem_limit_bytes=100 * 1024 * 1024,
    )

    body = _make_body(m1, bias is not None)
    if bias is not None:
        b_t = jnp.transpose(bias, (1, 0))
        b_spec = pl.BlockSpec((m2, d), lambda jn: (0, 0))
        out_t = pl.pallas_call(
            body,
            out_shape=out_shape,
            grid=(gn,),
            in_specs=[x_spec, w_spec, b_spec],
            out_specs=o_spec,
            compiler_params=cparams,
            cost_estimate=cost,
        )(x_t, w_t, b_t)
    else:
        out_t = pl.pallas_call(
            body,
            out_shape=out_shape,
            grid=(gn,),
            in_specs=[x_spec, w_spec],
            out_specs=o_spec,
            compiler_params=cparams,
            cost_estimate=cost,
        )(x_t, w_t)

    return jnp.transpose(out_t, (0, 2, 1))[:n]
